# baseline (speedup 1.0000x reference)
import functools

import jax
import jax.numpy as jnp
from jax import lax
from jax.experimental import pallas as pl
from jax.experimental.pallas import tpu as pltpu
from jax.experimental.pallas import tpu_sc as plsc

N = 10000
D = 128
T = 3
ET = 4
EDGES = 80000

NC = 2
NS = 16
CHUNK = 64
PER_CORE_RAW = ET * EDGES // NC
PER_TILE = 10240
NCHUNK = PER_TILE // CHUNK
SUPER = 8
NSUPER = NCHUNK // SUPER
DEPTH = 4
PER_CORE = PER_TILE * NS
ACC_ROWS = 10240
ZROWS = 16
BR = 400


def _messages_body(h_ref, w_ref, b_ref, out_ref):
    out_ref[0] = (
        jnp.dot(h_ref[...], w_ref[0], preferred_element_type=jnp.float32)
        + b_ref[0, 0]
    )


def _messages(h, w, b):
    return pl.pallas_call(
        _messages_body,
        grid=(ET, N // BR),
        in_specs=[
            pl.BlockSpec((BR, D), lambda e, r: (r, 0)),
            pl.BlockSpec((1, D, D), lambda e, r: (e, 0, 0)),
            pl.BlockSpec((1, 1, D), lambda e, r: (e, 0, 0)),
        ],
        out_specs=pl.BlockSpec((1, BR, D), lambda e, r: (e, r, 0)),
        out_shape=jax.ShapeDtypeStruct((ET, N, D), jnp.float32),
    )(h, w, b)


def _gru_math(p_ref, h_ref, wih_ref, whh_ref, bih_ref, bhh_ref):
    msum = p_ref[0] + p_ref[1]
    h = h_ref[...]
    gi = jnp.dot(msum, wih_ref[...], preferred_element_type=jnp.float32) + bih_ref[0]
    gh = jnp.dot(h, whh_ref[...], preferred_element_type=jnp.float32) + bhh_ref[0]
    r = jax.nn.sigmoid(gi[:, :D] + gh[:, :D])
    z = jax.nn.sigmoid(gi[:, D:2 * D] + gh[:, D:2 * D])
    n = jnp.tanh(gi[:, 2 * D:] + r * gh[:, 2 * D:])
    return (1.0 - z) * n + z * h


def _gru_m_body(p_ref, h_ref, wih_ref, whh_ref, bih_ref, bhh_ref,
                wn_ref, bn_ref, hout_ref, mout_ref):
    hn = _gru_math(p_ref, h_ref, wih_ref, whh_ref, bih_ref, bhh_ref)
    hout_ref[...] = hn
    for e in range(ET):
        mout_ref[e] = (
            jnp.dot(hn, wn_ref[e], preferred_element_type=jnp.float32)
            + bn_ref[e]
        )


def _gru_body(p_ref, h_ref, wih_ref, whh_ref, bih_ref, bhh_ref, hout_ref):
    hout_ref[...] = _gru_math(p_ref, h_ref, wih_ref, whh_ref, bih_ref, bhh_ref)


_GRU_IN_SPECS = [
    pl.BlockSpec((2, BR, D), lambda r: (0, r, 0)),
    pl.BlockSpec((BR, D), lambda r: (r, 0)),
    pl.BlockSpec((D, 3 * D), lambda r: (0, 0)),
    pl.BlockSpec((D, 3 * D), lambda r: (0, 0)),
    pl.BlockSpec((1, 3 * D), lambda r: (0, 0)),
    pl.BlockSpec((1, 3 * D), lambda r: (0, 0)),
]


def _gru_m(parts, h, wih_t, whh_t, b_ih2, b_hh2, wn, bn):
    return pl.pallas_call(
        _gru_m_body,
        grid=(N // BR,),
        in_specs=_GRU_IN_SPECS + [
            pl.BlockSpec((ET, D, D), lambda r: (0, 0, 0)),
            pl.BlockSpec((ET, 1, D), lambda r: (0, 0, 0)),
        ],
        out_specs=[
            pl.BlockSpec((BR, D), lambda r: (r, 0)),
            pl.BlockSpec((ET, BR, D), lambda r: (0, r, 0)),
        ],
        out_shape=[
            jax.ShapeDtypeStruct((N, D), jnp.float32),
            jax.ShapeDtypeStruct((ET, N, D), jnp.float32),
        ],
    )(parts, h, wih_t, whh_t, b_ih2, b_hh2, wn, bn)


def _gru_last(parts, h, wih_t, whh_t, b_ih2, b_hh2):
    return pl.pallas_call(
        _gru_body,
        grid=(N // BR,),
        in_specs=_GRU_IN_SPECS,
        out_specs=pl.BlockSpec((BR, D), lambda r: (r, 0)),
        out_shape=jax.ShapeDtypeStruct((N, D), jnp.float32),
    )(parts, h, wih_t, whh_t, b_ih2, b_hh2)


def _sc_segment_sum(m2d, src_all, dst_all):
    mesh = plsc.VectorSubcoreMesh(
        core_axis_name="c", subcore_axis_name="s",
        num_cores=NC, num_subcores=NS,
    )

    @functools.partial(
        pl.kernel,
        out_type=jax.ShapeDtypeStruct((NC, ACC_ROWS, D), jnp.float32),
        mesh=mesh,
        scratch_types=[
            pltpu.VMEM((PER_TILE,), jnp.int32),
            pltpu.VMEM((SUPER, CHUNK), jnp.int32),
            pltpu.VMEM((SUPER, CHUNK), jnp.int32),
            pltpu.VMEM((CHUNK, D), jnp.float32),
            pltpu.VMEM((CHUNK, D), jnp.float32),
            pltpu.VMEM((CHUNK, D), jnp.float32),
            pltpu.VMEM((CHUNK, D), jnp.float32),
            pltpu.VMEM((ZROWS, D), jnp.float32),
            pltpu.VMEM_SHARED((ACC_ROWS, D), jnp.float32),
            pltpu.SemaphoreType.DMA,
            pltpu.SemaphoreType.DMA,
            pltpu.SemaphoreType.DMA,
            pltpu.SemaphoreType.DMA,
            pltpu.SemaphoreType.DMA,
            pltpu.SemaphoreType.DMA,
        ],
    )
    def sc_kernel(m_hbm, src_hbm, dst_hbm, out_hbm,
                  sidx, didxa, didxb, rows0, rows1, rows2, rows3, zbuf, acc,
                  sem0, sem1, sem2, sem3, semi, semz):
        c = lax.axis_index("c")
        s = lax.axis_index("s")
        tile = c * NS + s
        ibase = tile * PER_TILE

        cp_s = pltpu.async_copy(
            src_hbm.at[pl.ds(ibase, PER_TILE)], sidx, sem0)
        cp_d = pltpu.async_copy(
            dst_hbm.at[pl.ds(tile * NCHUNK, SUPER)], didxa, semi)

        zero16 = jnp.zeros((16,), jnp.float32)
        for i in range(ZROWS):
            for j in range(D // 16):
                zbuf[i, pl.ds(j * 16, 16)] = zero16
        zrows_per_tile = ACC_ROWS // NS
        zbase = s * zrows_per_tile
        zcopies = [
            pltpu.async_copy(
                zbuf, acc.at[pl.ds(zbase + i * ZROWS, ZROWS)], semz)
            for i in range(zrows_per_tile // ZROWS)
        ]
        cp_s.wait()
        cp_d.wait()
        for cp in zcopies:
            cp.wait()
        plsc.subcore_barrier()

        rows = (rows0, rows1, rows2, rows3)
        sems = (sem0, sem1, sem2, sem3)

        def fire(j, p):
            pltpu.async_copy(
                m_hbm.at[sidx.at[pl.ds(j * CHUNK, CHUNK)]], rows[p],
                sems[p])

        for j in range(DEPTH):
            fire(j, j)

        def super_body(g, carry):
            not_last = g + 1 < NSUPER

            @pl.when(not_last)
            def _():
                pltpu.async_copy(
                    dst_hbm.at[pl.ds(tile * NCHUNK + (g + 1) * SUPER,
                                     SUPER)],
                    didxb, semi)

            for b in range(SUPER):
                p = b % DEPTH
                j = g * SUPER + b
                pltpu.make_async_copy(
                    m_hbm.at[sidx.at[pl.ds(b * CHUNK, CHUNK)]], rows[p],
                    sems[p]).wait()
                pltpu.sync_copy(rows[p], acc.at[didxa.at[b]], add=True)

                @pl.when(j + DEPTH < NCHUNK)
                def _():
                    fire(j + DEPTH, p)

            @pl.when(not_last)
            def _():
                pltpu.make_async_copy(
                    dst_hbm.at[pl.ds(tile * NCHUNK, SUPER)], didxb,
                    semi).wait()
                for r in range(SUPER):
                    for k in range(CHUNK // 16):
                        didxa[r, pl.ds(k * 16, 16)] = \
                            didxb[r, pl.ds(k * 16, 16)]

            return carry

        lax.fori_loop(0, NSUPER, super_body, 0)
        plsc.subcore_barrier()

        orows = ACC_ROWS // NS
        obase = s * orows
        pltpu.sync_copy(acc.at[pl.ds(obase, orows)],
                        out_hbm.at[c, pl.ds(obase, orows)])

    return sc_kernel(m2d, src_all, dst_all)


def _pad_edges(edges):
    srcs, dsts = [], []
    for e, ei in enumerate(edges):
        order = jnp.argsort(ei[1])
        dsts.append(ei[0][order])
        srcs.append(ei[1][order] + e * N)
    half = ET // NC
    src_parts, dst_parts = [], []
    pad = PER_CORE - PER_CORE_RAW
    for c in range(NC):
        s = jnp.concatenate(srcs[c * half:(c + 1) * half])
        d = jnp.concatenate(dsts[c * half:(c + 1) * half])
        src_parts.append(jnp.pad(s, (0, pad), constant_values=0))
        dst_parts.append(jnp.pad(d, (0, pad), constant_values=N))
    return (jnp.concatenate(src_parts),
            jnp.concatenate(dst_parts).reshape(-1, CHUNK))


@jax.jit
def kernel(x, edge_index_0, edge_index_1, edge_index_2, edge_index_3,
           weight, bias, w_ih, w_hh, b_ih, b_hh):
    src_all, dst_all = _pad_edges(
        [edge_index_0, edge_index_1, edge_index_2, edge_index_3])
    wih_t = w_ih.T
    whh_t = w_hh.T
    b_ih2 = b_ih.reshape(1, 3 * D)
    b_hh2 = b_hh.reshape(1, 3 * D)
    bias3 = bias.reshape(T, ET, 1, D)

    h = x
    m = _messages(x, weight[0], bias3[0])
    for t in range(T):
        parts = _sc_segment_sum(m.reshape(ET * N, D), src_all, dst_all)
        if t < T - 1:
            h, m = _gru_m(parts, h, wih_t, whh_t, b_ih2, b_hh2,
                          weight[t + 1], bias3[t + 1])
        else:
            h = _gru_last(parts, h, wih_t, whh_t, b_ih2, b_hh2)
    return h


# speedup vs baseline: 1.4775x; 1.4775x over previous
import functools

import jax
import jax.numpy as jnp
from jax import lax
from jax.experimental import pallas as pl
from jax.experimental.pallas import tpu as pltpu
from jax.experimental.pallas import tpu_sc as plsc

N = 10000
D = 128
T = 3
ET = 4
EDGES = 80000

NC = 2
NS = 16
CHUNK = 64
PER_CORE_RAW = ET * EDGES // NC
PER_TILE = 10240
NCHUNK = PER_TILE // CHUNK
SUPER = 8
NSUPER = NCHUNK // SUPER
DEPTH = 4
PER_CORE = PER_TILE * NS
ACC_ROWS = 10240
ZROWS = 16
BR = 400


def _messages_body(h_ref, w_ref, b_ref, out_ref):
    out_ref[0] = (
        jnp.dot(h_ref[...], w_ref[0], preferred_element_type=jnp.float32)
        + b_ref[0, 0]
    )


def _messages(h, w, b):
    return pl.pallas_call(
        _messages_body,
        grid=(ET, N // BR),
        in_specs=[
            pl.BlockSpec((BR, D), lambda e, r: (r, 0)),
            pl.BlockSpec((1, D, D), lambda e, r: (e, 0, 0)),
            pl.BlockSpec((1, 1, D), lambda e, r: (e, 0, 0)),
        ],
        out_specs=pl.BlockSpec((1, BR, D), lambda e, r: (e, r, 0)),
        out_shape=jax.ShapeDtypeStruct((ET, N, D), jnp.float32),
    )(h, w, b)


def _gru_math(p_ref, h_ref, wih_ref, whh_ref, bih_ref, bhh_ref):
    msum = p_ref[0] + p_ref[1]
    h = h_ref[...]
    gi = jnp.dot(msum, wih_ref[...], preferred_element_type=jnp.float32) + bih_ref[0]
    gh = jnp.dot(h, whh_ref[...], preferred_element_type=jnp.float32) + bhh_ref[0]
    r = jax.nn.sigmoid(gi[:, :D] + gh[:, :D])
    z = jax.nn.sigmoid(gi[:, D:2 * D] + gh[:, D:2 * D])
    n = jnp.tanh(gi[:, 2 * D:] + r * gh[:, 2 * D:])
    return (1.0 - z) * n + z * h


def _gru_m_body(p_ref, h_ref, wih_ref, whh_ref, bih_ref, bhh_ref,
                wn_ref, bn_ref, hout_ref, mout_ref):
    hn = _gru_math(p_ref, h_ref, wih_ref, whh_ref, bih_ref, bhh_ref)
    hout_ref[...] = hn
    for e in range(ET):
        mout_ref[e] = (
            jnp.dot(hn, wn_ref[e], preferred_element_type=jnp.float32)
            + bn_ref[e]
        )


def _gru_body(p_ref, h_ref, wih_ref, whh_ref, bih_ref, bhh_ref, hout_ref):
    hout_ref[...] = _gru_math(p_ref, h_ref, wih_ref, whh_ref, bih_ref, bhh_ref)


_GRU_IN_SPECS = [
    pl.BlockSpec((2, BR, D), lambda r: (0, r, 0)),
    pl.BlockSpec((BR, D), lambda r: (r, 0)),
    pl.BlockSpec((D, 3 * D), lambda r: (0, 0)),
    pl.BlockSpec((D, 3 * D), lambda r: (0, 0)),
    pl.BlockSpec((1, 3 * D), lambda r: (0, 0)),
    pl.BlockSpec((1, 3 * D), lambda r: (0, 0)),
]


def _gru_m(parts, h, wih_t, whh_t, b_ih2, b_hh2, wn, bn):
    return pl.pallas_call(
        _gru_m_body,
        grid=(N // BR,),
        in_specs=_GRU_IN_SPECS + [
            pl.BlockSpec((ET, D, D), lambda r: (0, 0, 0)),
            pl.BlockSpec((ET, 1, D), lambda r: (0, 0, 0)),
        ],
        out_specs=[
            pl.BlockSpec((BR, D), lambda r: (r, 0)),
            pl.BlockSpec((ET, BR, D), lambda r: (0, r, 0)),
        ],
        out_shape=[
            jax.ShapeDtypeStruct((N, D), jnp.float32),
            jax.ShapeDtypeStruct((ET, N, D), jnp.float32),
        ],
    )(parts, h, wih_t, whh_t, b_ih2, b_hh2, wn, bn)


def _gru_last(parts, h, wih_t, whh_t, b_ih2, b_hh2):
    return pl.pallas_call(
        _gru_body,
        grid=(N // BR,),
        in_specs=_GRU_IN_SPECS,
        out_specs=pl.BlockSpec((BR, D), lambda r: (r, 0)),
        out_shape=jax.ShapeDtypeStruct((N, D), jnp.float32),
    )(parts, h, wih_t, whh_t, b_ih2, b_hh2)


def _sc_segment_sum(m2d, src_all, dst_all):
    mesh = plsc.VectorSubcoreMesh(
        core_axis_name="c", subcore_axis_name="s",
        num_cores=NC, num_subcores=NS,
    )

    @functools.partial(
        pl.kernel,
        out_type=jax.ShapeDtypeStruct((NC, ACC_ROWS, D), jnp.float32),
        mesh=mesh,
        scratch_types=[
            pltpu.VMEM((PER_TILE,), jnp.int32),
            pltpu.VMEM((SUPER, CHUNK), jnp.int32),
            pltpu.VMEM((SUPER, CHUNK), jnp.int32),
            pltpu.VMEM((CHUNK, D), jnp.float32),
            pltpu.VMEM((CHUNK, D), jnp.float32),
            pltpu.VMEM((CHUNK, D), jnp.float32),
            pltpu.VMEM((CHUNK, D), jnp.float32),
            pltpu.VMEM((ZROWS, D), jnp.float32),
            pltpu.VMEM_SHARED((ACC_ROWS, D), jnp.float32),
            pltpu.SemaphoreType.DMA,
            pltpu.SemaphoreType.DMA,
            pltpu.SemaphoreType.DMA,
            pltpu.SemaphoreType.DMA,
            pltpu.SemaphoreType.DMA,
            pltpu.SemaphoreType.DMA,
        ],
    )
    def sc_kernel(m_hbm, src_hbm, dst_hbm, out_hbm,
                  sidx, didxa, didxb, rows0, rows1, rows2, rows3, zbuf, acc,
                  sem0, sem1, sem2, sem3, semi, semz):
        c = lax.axis_index("c")
        s = lax.axis_index("s")
        tile = c * NS + s
        ibase = tile * PER_TILE

        cp_s = pltpu.async_copy(
            src_hbm.at[pl.ds(ibase, PER_TILE)], sidx, sem0)
        cp_d = pltpu.async_copy(
            dst_hbm.at[pl.ds(tile * NCHUNK, SUPER)], didxa, semi)

        zero16 = jnp.zeros((16,), jnp.float32)
        for i in range(ZROWS):
            for j in range(D // 16):
                zbuf[i, pl.ds(j * 16, 16)] = zero16
        zrows_per_tile = ACC_ROWS // NS
        zbase = s * zrows_per_tile
        zcopies = [
            pltpu.async_copy(
                zbuf, acc.at[pl.ds(zbase + i * ZROWS, ZROWS)], semz)
            for i in range(zrows_per_tile // ZROWS)
        ]
        cp_s.wait()
        cp_d.wait()
        for cp in zcopies:
            cp.wait()
        plsc.subcore_barrier()

        rows = (rows0, rows1, rows2, rows3)
        sems = (sem0, sem1, sem2, sem3)

        def fire(j, p):
            pltpu.async_copy(
                m_hbm.at[sidx.at[pl.ds(j * CHUNK, CHUNK)]], rows[p],
                sems[p])

        for j in range(DEPTH):
            fire(j, j)

        def super_body(g, carry):
            not_last = g + 1 < NSUPER

            @pl.when(not_last)
            def _():
                pltpu.async_copy(
                    dst_hbm.at[pl.ds(tile * NCHUNK + (g + 1) * SUPER,
                                     SUPER)],
                    didxb, semi)

            for b in range(SUPER):
                p = b % DEPTH
                j = g * SUPER + b
                pltpu.make_async_copy(
                    m_hbm.at[sidx.at[pl.ds(b * CHUNK, CHUNK)]], rows[p],
                    sems[p]).wait()
                pltpu.sync_copy(rows[p], acc.at[didxa.at[b]], add=True)

                @pl.when(j + DEPTH < NCHUNK)
                def _():
                    fire(j + DEPTH, p)

            @pl.when(not_last)
            def _():
                pltpu.make_async_copy(
                    dst_hbm.at[pl.ds(tile * NCHUNK, SUPER)], didxb,
                    semi).wait()
                for r in range(SUPER):
                    for k in range(CHUNK // 16):
                        didxa[r, pl.ds(k * 16, 16)] = \
                            didxb[r, pl.ds(k * 16, 16)]

            return carry

        lax.fori_loop(0, NSUPER, super_body, 0)
        plsc.subcore_barrier()

        orows = ACC_ROWS // NS
        obase = s * orows
        pltpu.sync_copy(acc.at[pl.ds(obase, orows)],
                        out_hbm.at[c, pl.ds(obase, orows)])

    return sc_kernel(m2d, src_all, dst_all)


def _pad_edges(edges):
    srcs, dsts = [], []
    for e, ei in enumerate(edges):
        dsts.append(ei[0])
        srcs.append(ei[1] + e * N)
    half = ET // NC
    src_parts, dst_parts = [], []
    pad = PER_CORE - PER_CORE_RAW
    for c in range(NC):
        s = jnp.concatenate(srcs[c * half:(c + 1) * half])
        d = jnp.concatenate(dsts[c * half:(c + 1) * half])
        src_parts.append(jnp.pad(s, (0, pad), constant_values=0))
        dst_parts.append(jnp.pad(d, (0, pad), constant_values=N))
    return (jnp.concatenate(src_parts),
            jnp.concatenate(dst_parts).reshape(-1, CHUNK))


@jax.jit
def kernel(x, edge_index_0, edge_index_1, edge_index_2, edge_index_3,
           weight, bias, w_ih, w_hh, b_ih, b_hh):
    src_all, dst_all = _pad_edges(
        [edge_index_0, edge_index_1, edge_index_2, edge_index_3])
    wih_t = w_ih.T
    whh_t = w_hh.T
    b_ih2 = b_ih.reshape(1, 3 * D)
    b_hh2 = b_hh.reshape(1, 3 * D)
    bias3 = bias.reshape(T, ET, 1, D)

    h = x
    m = _messages(x, weight[0], bias3[0])
    for t in range(T):
        parts = _sc_segment_sum(m.reshape(ET * N, D), src_all, dst_all)
        if t < T - 1:
            h, m = _gru_m(parts, h, wih_t, whh_t, b_ih2, b_hh2,
                          weight[t + 1], bias3[t + 1])
        else:
            h = _gru_last(parts, h, wih_t, whh_t, b_ih2, b_hh2)
    return h
